# baseline (speedup 1.0000x reference)
import functools

import jax
import jax.numpy as jnp
from jax import lax
from jax.experimental import pallas as pl
from jax.experimental.pallas import tpu as pltpu
from jax.experimental.pallas import tpu_sc as plsc

NUM_CORES = 2
NUM_SUBCORES = 16
NW = NUM_CORES * NUM_SUBCORES

SUB = 128
G = 512
N_SUB = G // SUB
NBUF = 3

BW = 512


def _transpose_table(weights):
    V, D = weights.shape
    n_blk = pl.cdiv(V, 2 * BW)
    v_pad = n_blk * 2 * BW

    def body(a_ref, b_ref, out_ref):
        out_ref[...] = jnp.concatenate([a_ref[...].T, b_ref[...].T], axis=1)

    w_t = weights.T
    tp = pl.pallas_call(
        body,
        grid=(n_blk,),
        in_specs=[
            pl.BlockSpec((D, BW), lambda i: (0, 2 * i)),
            pl.BlockSpec((D, BW), lambda i: (0, 2 * i + 1)),
        ],
        out_specs=pl.BlockSpec((BW, 2 * D), lambda i: (i, 0)),
        out_shape=jax.ShapeDtypeStruct((v_pad // 2, 2 * D), jnp.float32),
    )(w_t, w_t)
    return tp.reshape(v_pad, D)


@functools.cache
def _build(B, VPAD, D):
    assert B % (NW * G) == 0
    b_per_w = B // NW
    n_gran = b_per_w // G
    mesh = plsc.VectorSubcoreMesh(core_axis_name="c", subcore_axis_name="s")

    @functools.partial(
        pl.kernel,
        mesh=mesh,
        out_type=jax.ShapeDtypeStruct((B, D), jnp.float32),
        scratch_types=[
            pltpu.VMEM((NBUF, G), jnp.int32),
            pltpu.VMEM((NBUF, G, D), jnp.float32),
            pltpu.SemaphoreType.DMA,
            pltpu.SemaphoreType.DMA,
            pltpu.SemaphoreType.DMA,
        ],
        compiler_params=pltpu.CompilerParams(use_tc_tiling_on_sc=False),
    )
    def gather_kernel(ids_hbm, table_hbm, out_hbm, idx_v, rows_v, sem_i,
                      sem_g, sem_w):
        wid = lax.axis_index("s") * NUM_CORES + lax.axis_index("c")
        base = wid * b_per_w

        def remap_idx(ib):
            for k in range(G // 16):
                v = idx_v[ib, pl.ds(16 * k, 16)]
                c = lax.bitwise_and(v, 1023)
                r = (v - c) + 2 * lax.bitwise_and(v, 511) \
                    + lax.bitwise_and(lax.shift_right_logical(v, 9), 1)
                idx_v[ib, pl.ds(16 * k, 16)] = r

        def fire_gathers(gb, ib):
            for j in range(N_SUB):
                pltpu.async_copy(
                    table_hbm.at[idx_v.at[ib, pl.ds(j * SUB, SUB)]],
                    rows_v.at[gb, pl.ds(j * SUB, SUB)],
                    sem_g,
                )

        def drain_gathers(gb):
            for j in range(N_SUB):
                pltpu.make_async_copy(
                    table_hbm.at[idx_v.at[0, pl.ds(j * SUB, SUB)]],
                    rows_v.at[gb, pl.ds(j * SUB, SUB)],
                    sem_g,
                ).wait()

        def stage_idx(g, ib, async_=True):
            src = ids_hbm.at[pl.ds(base + g * G, G)]
            if async_:
                pltpu.async_copy(src, idx_v.at[ib], sem_i)
            else:
                pltpu.sync_copy(src, idx_v.at[ib])

        def drain_idx():
            pltpu.make_async_copy(
                ids_hbm.at[pl.ds(base, G)], idx_v.at[0], sem_i
            ).wait()

        def start_write(g, gb):
            pltpu.async_copy(
                rows_v.at[gb], out_hbm.at[pl.ds(base + g * G, G)], sem_w
            )

        def drain_write(gb):
            pltpu.make_async_copy(
                rows_v.at[gb], out_hbm.at[pl.ds(base, G)], sem_w
            ).wait()

        stage_idx(0, 0, async_=False)
        remap_idx(0)
        fire_gathers(0, 0)
        stage_idx(1, 1)

        def loop_body(g, carry):
            b = lax.rem(g, NBUF)
            nb = lax.rem(g + 1, NBUF)

            @pl.when(g + 1 < n_gran)
            def _fire_next():
                drain_idx()
                remap_idx(nb)

                @pl.when(g >= 2)
                def _reclaim():
                    drain_write(nb)

                fire_gathers(nb, nb)

            drain_gathers(b)

            @pl.when(g + 2 < n_gran)
            def _prefetch_idx():
                stage_idx(g + 2, lax.rem(g + 2, NBUF))

            start_write(g, b)
            return carry

        lax.fori_loop(0, n_gran, loop_body, 0)

        for t in range(min(NBUF, n_gran)):
            drain_write(t)

    return gather_kernel


def kernel(token_ids, weights):
    B0, S = token_ids.shape
    V, D = weights.shape
    B = B0 * S
    table = _transpose_table(weights)
    ids = token_ids.reshape(B).astype(jnp.int32)
    out = _build(B, table.shape[0], D)(ids, table)
    return out.reshape(B0, S, D)
